# Initial kernel scaffold; baseline (speedup 1.0000x reference)
import jax
import jax.numpy as jnp
from jax.experimental import pallas as pl


def kernel(x):
    raise NotImplementedError("write your pallas kernel here")


# speedup vs baseline: 10.9019x; 10.9019x over previous
import jax
import jax.numpy as jnp
from jax.experimental import pallas as pl

N_ROWS = 16384
N_COLS = 4096
TILE_C = 128
K = (N_ROWS - 1) // 2

def _to_key(f):
    s = jax.lax.bitcast_convert_type(f, jnp.int32)
    return s ^ ((s >> 31) & 0x7FFFFFFF)


def _from_key(k):
    s = k ^ ((k >> 31) & 0x7FFFFFFF)
    return jax.lax.bitcast_convert_type(s, jnp.float32)


def _median_pair_body(x_ref, o_ref):
    key = _to_key(x_ref[...])

    def step(i, p):
        q = p + jax.lax.shift_left(jnp.ones((), jnp.int32), (31 - i).astype(jnp.int32))
        cnt = jnp.sum((key < q).astype(jnp.int32), axis=0, keepdims=True)
        return jnp.where(cnt <= K, q, p)

    p0 = jnp.full((1, TILE_C), -2147483648, dtype=jnp.int32)
    key_lo = jax.lax.fori_loop(0, 32, step, p0)

    cnt_le = jnp.sum((key <= key_lo).astype(jnp.int32), axis=0, keepdims=True)
    above = jnp.where(key > key_lo, key, 2147483647)
    key_hi = jnp.where(cnt_le >= K + 2, key_lo, jnp.min(above, axis=0, keepdims=True))

    lower = _from_key(key_lo)
    upper = _from_key(key_hi)
    o_ref[...] = jnp.abs(lower - (lower + upper) * 0.5)


@jax.jit
def kernel(x):
    out2d = pl.pallas_call(
        _median_pair_body,
        grid=(N_COLS // TILE_C,),
        in_specs=[pl.BlockSpec((N_ROWS, TILE_C), lambda i: (0, i))],
        out_specs=pl.BlockSpec((1, TILE_C), lambda i: (0, i)),
        out_shape=jax.ShapeDtypeStruct((1, N_COLS), jnp.float32),
    )(x)
    return out2d[0]


# speedup vs baseline: 28.4340x; 2.6082x over previous
import jax
import jax.numpy as jnp
from jax.experimental import pallas as pl

N_ROWS = 16384
N_COLS = 4096
TILE_C = 128
K = (N_ROWS - 1) // 2


def _to_key(f):
    s = jax.lax.bitcast_convert_type(f, jnp.int32)
    return s ^ ((s >> 31) & 0x7FFFFFFF)


def _from_key(k):
    s = k ^ ((k >> 31) & 0x7FFFFFFF)
    return jax.lax.bitcast_convert_type(s, jnp.float32)


def _median_pair_body(x_ref, o_ref):
    key = _to_key(x_ref[...])
    ktop = (key >> 16).astype(jnp.int16)
    klow = ((key & 0xFFFF) ^ 0x8000).astype(jnp.int16)

    ones_row = jnp.ones((1, N_ROWS), dtype=jnp.bfloat16)

    def count_below(vals, q):
        mask = jnp.where(vals < q, jnp.bfloat16(1), jnp.bfloat16(0))
        return jnp.dot(ones_row, mask, preferred_element_type=jnp.float32)

    kf = jnp.float32(K)

    def step_a(i, p):
        bit = jax.lax.shift_left(jnp.ones((), jnp.int32), 15 - i)
        q = p + bit
        return jnp.where(count_below(ktop, q.astype(jnp.int16)) <= kf, q, p)

    p16 = jax.lax.fori_loop(
        0, 16, step_a, jnp.full((1, TILE_C), -32768, dtype=jnp.int32))

    p16_16 = p16.astype(jnp.int16)
    c0 = count_below(ktop, p16_16)
    mlow = jnp.where(ktop == p16_16, klow, jnp.int16(32767))

    kb = kf - c0

    def step_b(i, p):
        bit = jax.lax.shift_left(jnp.ones((), jnp.int32), 15 - i)
        q = p + bit
        return jnp.where(count_below(mlow, q.astype(jnp.int16)) <= kb, q, p)

    plow = jax.lax.fori_loop(
        0, 16, step_b, jnp.full((1, TILE_C), -32768, dtype=jnp.int32))

    key_lo = (p16 << 16) | ((plow & 0xFFFF) ^ 0x8000)

    le_mask = jnp.where(key <= key_lo, 1.0, 0.0).astype(jnp.bfloat16)
    cnt_le = jnp.dot(ones_row, le_mask, preferred_element_type=jnp.float32)
    above = jnp.where(key > key_lo, key, 2147483647)
    key_hi = jnp.where(cnt_le >= jnp.float32(K + 2), key_lo,
                       jnp.min(above, axis=0, keepdims=True))

    lower = _from_key(key_lo)
    upper = _from_key(key_hi)
    o_ref[...] = jnp.abs(lower - (lower + upper) * 0.5)


@jax.jit
def kernel(x):
    out2d = pl.pallas_call(
        _median_pair_body,
        grid=(N_COLS // TILE_C,),
        in_specs=[pl.BlockSpec((N_ROWS, TILE_C), lambda i: (0, i))],
        out_specs=pl.BlockSpec((1, TILE_C), lambda i: (0, i)),
        out_shape=jax.ShapeDtypeStruct((1, N_COLS), jnp.float32),
    )(x)
    return out2d[0]
